# baseline (speedup 1.0000x reference)
import functools

import jax
import jax.numpy as jnp
from jax import lax
from jax.experimental import pallas as pl
from jax.experimental.pallas import tpu as pltpu
from jax.experimental.pallas import tpu_sc as plsc

B, L = 4096, 200
POI_DIM = 128
HOUR_DIM = 16
WKND_DIM = 8
USER_DIM = 64
HW_DIM = HOUR_DIM + WKND_DIM
OUT_DIM = POI_DIM + HW_DIM + USER_DIM
CT = OUT_DIM // 8
BT = B // 128

NUM_CORES = 2
NUM_SUBCORES = 16
NW = NUM_CORES * NUM_SUBCORES

STRIPS = tuple(
    [(0, 16 * k, 16 * k) for k in range(8)]
    + [(1, 0, 128), (1, 8, 136)]
    + [(2, 16 * k, 152 + 16 * k) for k in range(4)]
)


def _emb_body(poi_hbm, hour_hbm, wknd_hbm, user_hbm,
              hw_tbl, u_tbl, out_hbm,
              h_idx, w_idx, u_idx, hw_idx, hw_rows, u_rows, p_rows, col,
              sem_i0, sem_i1, sem_g0, sem_g1, sem_w0, sem_w1):
    wid = lax.axis_index("s") * NUM_CORES + lax.axis_index("c")
    bsl = pl.ds(wid * 128, 128)
    sem_i = (sem_i0, sem_i1)
    sem_g = (sem_g0, sem_g1)
    sem_w = (sem_w0, sem_w1)

    lane = lax.iota(jnp.int32, 16)
    strip_pat = []
    for (_, _, fbase) in STRIPS:
        c = lane + fbase
        strip_pat.append(lax.shift_right_logical(c, 3) * 1024
                         + lax.bitwise_and(c, 7) * 128)

    def fire_idx(l, p):
        pltpu.async_copy(hour_hbm.at[l, bsl], h_idx.at[p], sem_i[p])
        pltpu.async_copy(wknd_hbm.at[l, bsl], w_idx.at[p], sem_i[p])
        pltpu.async_copy(user_hbm.at[l, bsl], u_idx.at[p], sem_i[p])

    def wait_idx(p):
        for dst in (h_idx, w_idx, u_idx):
            pltpu.make_async_copy(hour_hbm.at[0, pl.ds(0, 128)],
                                  dst.at[p], sem_i[p]).wait()

    def compute_hw(p):
        for k in range(8):
            sl = pl.ds(k * 16, 16)
            hw_idx[p, sl] = h_idx[p, sl] * 3 + w_idx[p, sl]

    def fire_pg(l, p):
        pltpu.async_copy(u_tbl.at[u_idx.at[p]], u_rows.at[p], sem_g[p])
        pltpu.async_copy(hw_tbl.at[hw_idx.at[p]], hw_rows.at[p], sem_g[p])
        pltpu.async_copy(poi_hbm.at[bsl, pl.ds(l, 1), :], p_rows.at[p],
                         sem_g[p])

    def wait_pg(p):
        pltpu.make_async_copy(u_tbl.at[pl.ds(0, 128)], u_rows.at[p],
                              sem_g[p]).wait()
        pltpu.make_async_copy(u_tbl.at[pl.ds(0, 128), pl.ds(0, HW_DIM)],
                              hw_rows.at[p], sem_g[p]).wait()
        pltpu.make_async_copy(poi_hbm.at[bsl, pl.ds(0, 1), :],
                              p_rows.at[p], sem_g[p]).wait()

    def transpose(p):
        def tok_body(t):
            tb = jnp.full((16,), t, jnp.int32)
            for i, (src, off, _) in enumerate(STRIPS):
                if src == 0:
                    x = p_rows[p, t, 0, pl.ds(off, 16)]
                elif src == 1:
                    x = hw_rows[p, t, pl.ds(off, 16)]
                else:
                    x = u_rows[p, t, pl.ds(off, 16)]
                plsc.store_scatter(col.at[p], [strip_pat[i] + tb], x)
        pl.loop(0, 128, unroll=2)(tok_body)

    def fire_write(l, p):
        base = (l * CT * BT + wid) * 1024
        for ct in range(CT):
            pltpu.async_copy(col.at[p, pl.ds(ct * 1024, 1024)],
                             out_hbm.at[pl.ds(base + ct * BT * 1024, 1024)],
                             sem_w[p])

    def drain_write(p):
        for _ in range(CT):
            pltpu.make_async_copy(col.at[p, pl.ds(0, 1024)],
                                  out_hbm.at[pl.ds(0, 1024)],
                                  sem_w[p]).wait()

    fire_idx(0, 0)
    wait_idx(0)
    compute_hw(0)
    fire_pg(0, 0)
    fire_idx(1, 1)

    def step(l, p):
        @pl.when(l + 1 < L)
        def _():
            wait_idx(1 - p)
            compute_hw(1 - p)
        wait_pg(p)
        @pl.when(l + 1 < L)
        def _():
            fire_pg(l + 1, 1 - p)
        @pl.when(l + 2 < L)
        def _():
            fire_idx(l + 2, p)
        @pl.when(l >= 2)
        def _():
            drain_write(p)
        transpose(p)
        fire_write(l, p)

    def pair_body(j):
        step(2 * j, 0)
        step(2 * j + 1, 1)
    pl.loop(0, L // 2)(pair_body)
    drain_write(0)
    drain_write(1)


_mesh = plsc.VectorSubcoreMesh(core_axis_name="c", subcore_axis_name="s")

_emb_kernel = functools.partial(
    pl.kernel,
    out_type=jax.ShapeDtypeStruct((L * CT * BT * 1024,), jnp.float32),
    mesh=_mesh,
    compiler_params=pltpu.CompilerParams(use_tc_tiling_on_sc=False,
                                         needs_layout_passes=False),
    scratch_types=[
        pltpu.VMEM((2, 128), jnp.int32),
        pltpu.VMEM((2, 128), jnp.int32),
        pltpu.VMEM((2, 128), jnp.int32),
        pltpu.VMEM((2, 128), jnp.int32),
        pltpu.VMEM((2, 128, HW_DIM), jnp.float32),
        pltpu.VMEM((2, 128, USER_DIM), jnp.float32),
        pltpu.VMEM((2, 128, 1, POI_DIM), jnp.float32),
        pltpu.VMEM((2, CT * 8 * 128), jnp.float32),
        pltpu.SemaphoreType.DMA,
        pltpu.SemaphoreType.DMA,
        pltpu.SemaphoreType.DMA,
        pltpu.SemaphoreType.DMA,
        pltpu.SemaphoreType.DMA,
        pltpu.SemaphoreType.DMA,
    ],
)(_emb_body)


@jax.jit
def kernel(seq_poi_embeddings, hour_set, isweekend_set, user_set,
           hour_table, isweekend_table, user_table):
    hour = hour_set.T
    wknd = isweekend_set.T
    user = user_set.T
    h_tbl = hour_table.at[0].set(0.0)
    w_tbl = isweekend_table.at[0].set(0.0)
    hw_tbl = jnp.concatenate(
        [jnp.broadcast_to(h_tbl[:, None, :], (25, 3, HOUR_DIM)),
         jnp.broadcast_to(w_tbl[None, :, :], (25, 3, WKND_DIM))],
        axis=2).reshape(75, HW_DIM)
    u_tbl = user_table.at[0].set(0.0)
    raw = _emb_kernel(seq_poi_embeddings, hour, wknd, user, hw_tbl, u_tbl)
    out5 = raw.reshape(L, CT, BT, 8, 128)
    return out5.transpose(2, 4, 0, 1, 3).reshape(B, L, OUT_DIM)


# speedup vs baseline: 1.1573x; 1.1573x over previous
import functools

import jax
import jax.numpy as jnp
from jax import lax
from jax.experimental import pallas as pl
from jax.experimental.pallas import tpu as pltpu
from jax.experimental.pallas import tpu_sc as plsc

B, L = 4096, 200
N = B * L
POI_DIM = 128
HOUR_DIM = 16
WKND_DIM = 8
USER_DIM = 64
HW_DIM = HOUR_DIM + WKND_DIM
TAIL_DIM = HW_DIM + USER_DIM
OUT_DIM = POI_DIM + TAIL_DIM

NUM_CORES = 2
NUM_SUBCORES = 16
NW = NUM_CORES * NUM_SUBCORES
ROWS_PER_W = B // NW
CB = 2
CHUNK = CB * L
NCHUNK = ROWS_PER_W // CB
IDX_GROUPS = ((0, 128), (128, 128), (256, 128), (384, 16))


def _emb_body(poi_hbm, hour_hbm, wknd_hbm, user_hbm,
              hw_tbl, u_tbl, out_hbm,
              h_idx, w_idx, u_idx, hw_idx, hw_rows, t_rows, p_rows, sem,
              sem_w):
    wid = lax.axis_index("s") * NUM_CORES + lax.axis_index("c")
    w_base = wid * ROWS_PER_W

    def chunk_body(i):
        b0 = w_base + i * CB
        base = b0 * L
        tsl = pl.ds(base, CHUNK)
        cps = [
            pltpu.async_copy(hour_hbm.at[tsl], h_idx, sem),
            pltpu.async_copy(wknd_hbm.at[tsl], w_idx, sem),
            pltpu.async_copy(user_hbm.at[tsl], u_idx, sem),
        ]
        cps_p = [
            pltpu.async_copy(poi_hbm.at[b0 + bl],
                             p_rows.at[pl.ds(bl * L, L)], sem_w)
            for bl in range(CB)
        ]
        for cp in cps:
            cp.wait()
        for k in range(CHUNK // 16):
            sl = pl.ds(k * 16, 16)
            hw_idx[sl] = h_idx[sl] * 3 + w_idx[sl]
        cps = []
        for (off, ln) in IDX_GROUPS:
            d = pl.ds(off, ln)
            cps.append(pltpu.async_copy(
                u_tbl.at[u_idx.at[d]], t_rows.at[d], sem))
            cps.append(pltpu.async_copy(
                hw_tbl.at[hw_idx.at[d]], hw_rows.at[d], sem))
        for cp in cps:
            cp.wait()

        def tok_body(t):
            t_rows[t, pl.ds(0, 16)] = hw_rows[t, pl.ds(0, 16)]
            t_rows[t, pl.ds(8, 16)] = hw_rows[t, pl.ds(8, 16)]
        pl.loop(0, CHUNK, unroll=8)(tok_body)

        for cp in cps_p:
            cp.wait()
        out = out_hbm.at[pl.ds(base, CHUNK)]
        cp1 = pltpu.async_copy(p_rows, out.at[:, pl.ds(0, POI_DIM)], sem_w)
        cp2 = pltpu.async_copy(t_rows, out.at[:, pl.ds(POI_DIM, TAIL_DIM)],
                               sem_w)
        cp1.wait()
        cp2.wait()

    pl.loop(0, NCHUNK)(chunk_body)


_mesh = plsc.VectorSubcoreMesh(core_axis_name="c", subcore_axis_name="s")

_emb_kernel = functools.partial(
    pl.kernel,
    out_type=jax.ShapeDtypeStruct((N, OUT_DIM), jnp.float32),
    mesh=_mesh,
    compiler_params=pltpu.CompilerParams(use_tc_tiling_on_sc=False),
    scratch_types=[
        pltpu.VMEM((CHUNK,), jnp.int32),
        pltpu.VMEM((CHUNK,), jnp.int32),
        pltpu.VMEM((CHUNK,), jnp.int32),
        pltpu.VMEM((CHUNK,), jnp.int32),
        pltpu.VMEM((CHUNK, HW_DIM), jnp.float32),
        pltpu.VMEM((CHUNK, TAIL_DIM), jnp.float32),
        pltpu.VMEM((CHUNK, POI_DIM), jnp.float32),
        pltpu.SemaphoreType.DMA,
        pltpu.SemaphoreType.DMA,
    ],
)(_emb_body)


@jax.jit
def kernel(seq_poi_embeddings, hour_set, isweekend_set, user_set,
           hour_table, isweekend_table, user_table):
    hour = hour_set.reshape(N)
    wknd = isweekend_set.reshape(N)
    user = user_set.reshape(N)
    h_tbl = hour_table.at[0].set(0.0)
    w_tbl = isweekend_table.at[0].set(0.0)
    hw_tbl = jnp.concatenate(
        [jnp.broadcast_to(h_tbl[:, None, :], (25, 3, HOUR_DIM)),
         jnp.broadcast_to(w_tbl[None, :, :], (25, 3, WKND_DIM))],
        axis=2).reshape(75, HW_DIM)
    u_tbl = jnp.concatenate(
        [jnp.zeros((100001, HW_DIM), jnp.float32),
         user_table.at[0].set(0.0)], axis=1)
    out = _emb_kernel(seq_poi_embeddings, hour, wknd, user, hw_tbl, u_tbl)
    return out.reshape(B, L, OUT_DIM)
